# baseline (device time: 12429 ns/iter reference)
import jax
import jax.numpy as jnp
from jax import lax
from jax.experimental import pallas as pl
from jax.experimental.pallas import tpu as pltpu

N_DEV = 16
B = 2
S = 128
HQ = 4
DH = 64
HD = HQ * DH
NEG = -1e9


def kernel(x, Wq, K_ext, V_ext, Wo):
    d_model = x.shape[-1]

    def body(x_ref, wq_ref, k_ref, v_ref, wo_ref, out_ref,
             kbuf, vbuf, send_sems, recv_sems):
        my = lax.axis_index("i")
        left = my - 1
        right = my + 1
        has_left = my > 0
        has_right = my < N_DEV - 1

        @pl.when(jnp.logical_not(has_left))
        def _():
            kbuf[0] = jnp.zeros((B, S, HD), jnp.bfloat16)
            vbuf[0] = jnp.zeros((B, S, HD), jnp.bfloat16)

        @pl.when(jnp.logical_not(has_right))
        def _():
            kbuf[1] = jnp.zeros((B, S, HD), jnp.bfloat16)
            vbuf[1] = jnp.zeros((B, S, HD), jnp.bfloat16)

        barrier_sem = pltpu.get_barrier_semaphore()

        @pl.when(has_left)
        def _():
            pl.semaphore_signal(barrier_sem, inc=1, device_id=(left,),
                                device_id_type=pl.DeviceIdType.MESH)

        @pl.when(has_right)
        def _():
            pl.semaphore_signal(barrier_sem, inc=1, device_id=(right,),
                                device_id_type=pl.DeviceIdType.MESH)

        n_nbr = has_left.astype(jnp.int32) + has_right.astype(jnp.int32)
        pl.semaphore_wait(barrier_sem, n_nbr)

        def halo_rdma(src, buf, slot, sem, nbr):
            return pltpu.make_async_remote_copy(
                src_ref=src,
                dst_ref=buf.at[slot],
                send_sem=send_sems.at[sem],
                recv_sem=recv_sems.at[sem],
                device_id=(nbr,),
                device_id_type=pl.DeviceIdType.MESH,
            )

        @pl.when(has_right)
        def _():
            halo_rdma(k_ref, kbuf, 0, 0, right).start()

        @pl.when(has_left)
        def _():
            halo_rdma(k_ref, kbuf, 1, 2, left).start()

        @pl.when(has_right)
        def _():
            halo_rdma(v_ref, vbuf, 0, 1, right).start()

        @pl.when(has_left)
        def _():
            halo_rdma(v_ref, vbuf, 1, 3, left).start()

        x2 = x_ref[...].reshape(B * S, x_ref.shape[-1])
        q2 = jnp.dot(x2, wq_ref[...],
                     preferred_element_type=jnp.float32)

        qi = lax.broadcasted_iota(jnp.int32, (S, S), 0)
        kj = lax.broadcasted_iota(jnp.int32, (S, S), 1)
        full_mask = jnp.concatenate(
            [jnp.logical_and(qi <= kj, has_left),
             jnp.ones((S, S), jnp.bool_),
             jnp.logical_and(qi >= kj, has_right)], axis=1)

        @pl.when(has_left)
        def _():
            halo_rdma(k_ref, kbuf, 0, 0, left).wait_recv()

        @pl.when(has_right)
        def _():
            halo_rdma(k_ref, kbuf, 1, 2, right).wait_recv()

        dn = (((1,), (1,)), ((), ()))
        w_all = []
        for b in range(B):
            k_full = jnp.concatenate(
                [kbuf[0, b], k_ref[b], kbuf[1, b]], axis=0)
            for h in range(HQ):
                hs = slice(h * DH, (h + 1) * DH)
                q = q2[b * S:(b + 1) * S, hs].astype(jnp.bfloat16)
                s = lax.dot_general(q, k_full[:, hs], dn,
                                    preferred_element_type=jnp.float32)
                w = jnp.where(full_mask, jnp.exp(s), 0.0)
                w = w / jnp.sum(w, axis=1, keepdims=True)
                w_all.append(w.astype(jnp.bfloat16))

        @pl.when(has_left)
        def _():
            halo_rdma(v_ref, vbuf, 0, 1, left).wait_recv()

        @pl.when(has_right)
        def _():
            halo_rdma(v_ref, vbuf, 1, 3, right).wait_recv()

        ctx_rows = []
        for b in range(B):
            v_full = jnp.concatenate(
                [vbuf[0, b], v_ref[b], vbuf[1, b]], axis=0)
            heads = [
                jnp.dot(w_all[b * HQ + h],
                        v_full[:, h * DH:(h + 1) * DH],
                        preferred_element_type=jnp.float32)
                for h in range(HQ)
            ]
            ctx_rows.append(jnp.concatenate(heads, axis=1))
        ctx = jnp.concatenate(ctx_rows, axis=0)
        out = jnp.dot(ctx.astype(jnp.bfloat16), wo_ref[...],
                      preferred_element_type=jnp.float32)
        for b in range(B):
            out_ref[b] = out[b * S:(b + 1) * S, :]

        @pl.when(has_right)
        def _():
            halo_rdma(k_ref, kbuf, 0, 0, right).wait_send()
            halo_rdma(v_ref, vbuf, 0, 1, right).wait_send()

        @pl.when(has_left)
        def _():
            halo_rdma(k_ref, kbuf, 1, 2, left).wait_send()
            halo_rdma(v_ref, vbuf, 1, 3, left).wait_send()

    k2 = K_ext.reshape(B, S, HD).astype(jnp.bfloat16)
    v2 = V_ext.reshape(B, S, HD).astype(jnp.bfloat16)
    xb = x.astype(jnp.bfloat16)
    wqs = (Wq * 0.125).astype(jnp.bfloat16)
    wob = Wo.astype(jnp.bfloat16)

    return pl.pallas_call(
        body,
        out_shape=jax.ShapeDtypeStruct((B, S, d_model), jnp.float32),
        in_specs=[pl.BlockSpec(memory_space=pltpu.VMEM)] * 5,
        out_specs=pl.BlockSpec(memory_space=pltpu.VMEM),
        scratch_shapes=[
            pltpu.VMEM((2, B, S, HD), jnp.bfloat16),
            pltpu.VMEM((2, B, S, HD), jnp.bfloat16),
            pltpu.SemaphoreType.DMA((4,)),
            pltpu.SemaphoreType.DMA((4,)),
        ],
        compiler_params=pltpu.CompilerParams(collective_id=0),
    )(xb, wqs, k2, v2, wob)


# device time: 11062 ns/iter; 1.1236x vs baseline; 1.1236x over previous
import jax
import jax.numpy as jnp
from jax import lax
from jax.experimental import pallas as pl
from jax.experimental.pallas import tpu as pltpu

N_DEV = 16
B = 2
S = 128
HQ = 4
DH = 64
HD = HQ * DH
NEG = -1e9


def kernel(x, Wq, K_ext, V_ext, Wo):
    d_model = x.shape[-1]

    def body(x_ref, wq_ref, k_ref, v_ref, wo_ref, out_ref,
             kbuf, vbuf, send_sems, recv_sems):
        my = lax.axis_index("i")
        left = my - 1
        right = my + 1
        has_left = my > 0
        has_right = my < N_DEV - 1

        @pl.when(jnp.logical_not(has_left))
        def _():
            kbuf[0] = jnp.zeros((B, S, HD), jnp.float8_e4m3fn)
            vbuf[0] = jnp.zeros((B, S, HD), jnp.float8_e4m3fn)

        @pl.when(jnp.logical_not(has_right))
        def _():
            kbuf[1] = jnp.zeros((B, S, HD), jnp.float8_e4m3fn)
            vbuf[1] = jnp.zeros((B, S, HD), jnp.float8_e4m3fn)

        barrier_sem = pltpu.get_barrier_semaphore()

        @pl.when(has_left)
        def _():
            pl.semaphore_signal(barrier_sem, inc=1, device_id=(left,),
                                device_id_type=pl.DeviceIdType.MESH)

        @pl.when(has_right)
        def _():
            pl.semaphore_signal(barrier_sem, inc=1, device_id=(right,),
                                device_id_type=pl.DeviceIdType.MESH)

        n_nbr = has_left.astype(jnp.int32) + has_right.astype(jnp.int32)
        pl.semaphore_wait(barrier_sem, n_nbr)

        def halo_rdma(src, buf, slot, sem, nbr):
            return pltpu.make_async_remote_copy(
                src_ref=src,
                dst_ref=buf.at[slot],
                send_sem=send_sems.at[sem],
                recv_sem=recv_sems.at[sem],
                device_id=(nbr,),
                device_id_type=pl.DeviceIdType.MESH,
            )

        @pl.when(has_right)
        def _():
            halo_rdma(k_ref, kbuf, 0, 0, right).start()

        @pl.when(has_left)
        def _():
            halo_rdma(k_ref, kbuf, 1, 2, left).start()

        @pl.when(has_right)
        def _():
            halo_rdma(v_ref, vbuf, 0, 1, right).start()

        @pl.when(has_left)
        def _():
            halo_rdma(v_ref, vbuf, 1, 3, left).start()

        x2 = x_ref[...].reshape(B * S, x_ref.shape[-1])
        q2 = jnp.dot(x2, wq_ref[...],
                     preferred_element_type=jnp.float32)

        qi = lax.broadcasted_iota(jnp.int32, (S, S), 0)
        kj = lax.broadcasted_iota(jnp.int32, (S, S), 1)
        full_mask = jnp.concatenate(
            [jnp.logical_and(qi <= kj, has_left),
             jnp.ones((S, S), jnp.bool_),
             jnp.logical_and(qi >= kj, has_right)], axis=1)

        @pl.when(has_left)
        def _():
            halo_rdma(k_ref, kbuf, 0, 0, left).wait_recv()

        @pl.when(has_right)
        def _():
            halo_rdma(k_ref, kbuf, 1, 2, right).wait_recv()

        dn = (((1,), (1,)), ((), ()))
        w_all = []
        for b in range(B):
            k_full = jnp.concatenate(
                [kbuf[0, b], k_ref[b], kbuf[1, b]],
                axis=0).astype(jnp.bfloat16)
            for h in range(HQ):
                hs = slice(h * DH, (h + 1) * DH)
                q = q2[b * S:(b + 1) * S, hs].astype(jnp.bfloat16)
                s = lax.dot_general(q, k_full[:, hs], dn,
                                    preferred_element_type=jnp.float32)
                w = jnp.where(full_mask, jnp.exp(s), 0.0)
                w = w / jnp.sum(w, axis=1, keepdims=True)
                w_all.append(w.astype(jnp.bfloat16))

        @pl.when(has_left)
        def _():
            halo_rdma(v_ref, vbuf, 0, 1, left).wait_recv()

        @pl.when(has_right)
        def _():
            halo_rdma(v_ref, vbuf, 1, 3, right).wait_recv()

        ctx_rows = []
        for b in range(B):
            v_full = jnp.concatenate(
                [vbuf[0, b], v_ref[b], vbuf[1, b]],
                axis=0).astype(jnp.bfloat16)
            heads = [
                jnp.dot(w_all[b * HQ + h],
                        v_full[:, h * DH:(h + 1) * DH],
                        preferred_element_type=jnp.float32)
                for h in range(HQ)
            ]
            ctx_rows.append(jnp.concatenate(heads, axis=1))
        ctx = jnp.concatenate(ctx_rows, axis=0)
        out = jnp.dot(ctx.astype(jnp.bfloat16), wo_ref[...],
                      preferred_element_type=jnp.float32)
        for b in range(B):
            out_ref[b] = out[b * S:(b + 1) * S, :]

        @pl.when(has_right)
        def _():
            halo_rdma(k_ref, kbuf, 0, 0, right).wait_send()
            halo_rdma(v_ref, vbuf, 0, 1, right).wait_send()

        @pl.when(has_left)
        def _():
            halo_rdma(k_ref, kbuf, 1, 2, left).wait_send()
            halo_rdma(v_ref, vbuf, 1, 3, left).wait_send()

    k2 = K_ext.reshape(B, S, HD).astype(jnp.float8_e4m3fn)
    v2 = V_ext.reshape(B, S, HD).astype(jnp.float8_e4m3fn)
    xb = x.astype(jnp.bfloat16)
    wqs = (Wq * 0.125).astype(jnp.bfloat16)
    wob = Wo.astype(jnp.bfloat16)

    return pl.pallas_call(
        body,
        out_shape=jax.ShapeDtypeStruct((B, S, d_model), jnp.float32),
        in_specs=[pl.BlockSpec(memory_space=pltpu.VMEM)] * 5,
        out_specs=pl.BlockSpec(memory_space=pltpu.VMEM),
        scratch_shapes=[
            pltpu.VMEM((2, B, S, HD), jnp.float8_e4m3fn),
            pltpu.VMEM((2, B, S, HD), jnp.float8_e4m3fn),
            pltpu.SemaphoreType.DMA((4,)),
            pltpu.SemaphoreType.DMA((4,)),
        ],
        compiler_params=pltpu.CompilerParams(collective_id=0),
    )(xb, wqs, k2, v2, wob)
